# baseline (speedup 1.0000x reference)
import functools

import jax
import jax.numpy as jnp
from jax import lax
from jax.experimental import pallas as pl
from jax.experimental.pallas import tpu as pltpu
from jax.experimental.pallas import tpu_sc as plsc

N = 10000
E = 320000
D_IN = 128
D_HID = 128
D_OUT = 64

NC = 2
NS = 16
NW = NC * NS

CHUNK = 128
CH = 80
E_PAD = NW * CH * CHUNK
N_ACC = 10240
ROWS_PER_TILE = N_ACC // NS

CB = 20
TOT_CHUNKS = E_PAD // CHUNK
B0 = 5
B1 = 3


def _spmm_body(h_hbm, src_hbm, dst_hbm, out_hbm,
               src_v, dst_v, rows_v, accum, gsem, ssem, zsem, D):
    c = lax.axis_index("c")
    s = lax.axis_index("s")

    nblk = lax.select(c == 0, B0, B1)
    base = lax.select(c == 0, s * (B0 * CB), NS * B0 * CB + s * (B1 * CB))

    def gather_start(q, slot):
        pltpu.async_copy(h_hbm.at[src_v.at[q]], rows_v.at[slot],
                         gsem.at[slot])

    def gather_wait(q, slot):
        pltpu.make_async_copy(h_hbm.at[src_v.at[q]], rows_v.at[slot],
                              gsem.at[slot]).wait()

    def scatter_start(q, slot):
        pltpu.async_copy(rows_v.at[slot], accum.at[dst_v.at[q]],
                         ssem.at[slot], add=True)

    def scatter_wait(q, slot):
        pltpu.make_async_copy(rows_v.at[slot], accum.at[dst_v.at[q]],
                              ssem.at[slot]).wait()

    zvec = jnp.zeros((16,), jnp.float32)
    for i in range(16):
        for jv in range(D // 16):
            rows_v[0, i, pl.ds(jv * 16, 16)] = zvec

    def zero_start(k, carry):
        pltpu.async_copy(rows_v.at[0, pl.ds(0, 16)],
                         accum.at[pl.ds(s * ROWS_PER_TILE + k * 16, 16)],
                         zsem)
        return carry

    def zero_wait(k, carry):
        pltpu.make_async_copy(rows_v.at[0, pl.ds(0, 16)],
                              accum.at[pl.ds(s * ROWS_PER_TILE + k * 16, 16)],
                              zsem).wait()
        return carry

    lax.fori_loop(0, ROWS_PER_TILE // 16, zero_start, 0)
    lax.fori_loop(0, ROWS_PER_TILE // 16, zero_wait, 0)
    plsc.subcore_barrier()

    def blk_body(blk, carry):
        pltpu.sync_copy(src_hbm.at[pl.ds(base + blk * CB, CB)], src_v)
        pltpu.sync_copy(dst_hbm.at[pl.ds(base + blk * CB, CB)], dst_v)

        gather_start(0, 0)
        gather_start(1, 1)
        gather_wait(0, 0)
        scatter_start(0, 0)

        def superstep(s2, carry2):
            for b in range(2):
                q = s2 * 2 + b
                scatter_wait(q - 2, b)
                gather_start(q, b)
                gather_wait(q - 1, 1 - b)
                scatter_start(q - 1, 1 - b)
            return carry2

        lax.fori_loop(1, CB // 2, superstep, 0)

        gather_wait(CB - 1, 1)
        scatter_start(CB - 1, 1)
        scatter_wait(CB - 2, 0)
        scatter_wait(CB - 1, 1)
        return carry

    lax.fori_loop(0, nblk, blk_body, 0)
    plsc.subcore_barrier()

    pltpu.sync_copy(
        accum.at[pl.ds(s * ROWS_PER_TILE, ROWS_PER_TILE)],
        out_hbm.at[c, pl.ds(s * ROWS_PER_TILE, ROWS_PER_TILE)])


def _make_spmm(D):
    mesh = plsc.VectorSubcoreMesh(core_axis_name="c", subcore_axis_name="s")
    body = functools.partial(_spmm_body, D=D)
    return pl.kernel(
        body,
        out_type=jax.ShapeDtypeStruct((NC, N_ACC, D), jnp.float32),
        mesh=mesh,
        scratch_types=[
            pltpu.VMEM((CB, CHUNK), jnp.int32),
            pltpu.VMEM((CB, CHUNK), jnp.int32),
            pltpu.VMEM((2, CHUNK, D), jnp.float32),
            pltpu.VMEM_SHARED((N_ACC, D), jnp.float32),
            pltpu.SemaphoreType.DMA((2,)),
            pltpu.SemaphoreType.DMA((2,)),
            pltpu.SemaphoreType.DMA,
        ],
        compiler_params=pltpu.CompilerParams(use_tc_tiling_on_sc=False),
        name=f"spmm_sc_d{D}",
    )


_spmm_l1 = _make_spmm(D_HID)
_spmm_l2 = _make_spmm(D_OUT)


_BM = 1000


def _mm1_body(x_ref, w_ref, o_ref):
    o_ref[...] = jnp.dot(x_ref[...], w_ref[...],
                         preferred_element_type=jnp.float32)


def _mid_body(p0_ref, p1_ref, w_ref, o_ref):
    h = jnp.maximum(p0_ref[0] + p1_ref[0], 0.0)
    o_ref[...] = jnp.dot(h, w_ref[...], preferred_element_type=jnp.float32)


def _final_body(p0_ref, p1_ref, o_ref):
    z = p0_ref[0] + p1_ref[0]
    m = jnp.max(z, axis=1, keepdims=True)
    lse = jnp.log(jnp.sum(jnp.exp(z - m), axis=1, keepdims=True)) + m
    o_ref[...] = z - lse


def _mm1(x, w1):
    return pl.pallas_call(
        _mm1_body,
        grid=(N // _BM,),
        in_specs=[
            pl.BlockSpec((_BM, D_IN), lambda i: (i, 0)),
            pl.BlockSpec((D_IN, D_HID), lambda i: (0, 0)),
        ],
        out_specs=pl.BlockSpec((_BM, D_HID), lambda i: (i, 0)),
        out_shape=jax.ShapeDtypeStruct((N, D_HID), jnp.float32),
    )(x, w1)


def _mid(parts, w2):
    return pl.pallas_call(
        _mid_body,
        grid=(N // _BM,),
        in_specs=[
            pl.BlockSpec((1, _BM, D_HID), lambda i: (0, i, 0)),
            pl.BlockSpec((1, _BM, D_HID), lambda i: (1, i, 0)),
            pl.BlockSpec((D_HID, D_OUT), lambda i: (0, 0)),
        ],
        out_specs=pl.BlockSpec((_BM, D_OUT), lambda i: (i, 0)),
        out_shape=jax.ShapeDtypeStruct((N, D_OUT), jnp.float32),
    )(parts, parts, w2)


def _final(parts):
    return pl.pallas_call(
        _final_body,
        grid=(N // _BM,),
        in_specs=[
            pl.BlockSpec((1, _BM, D_OUT), lambda i: (0, i, 0)),
            pl.BlockSpec((1, _BM, D_OUT), lambda i: (1, i, 0)),
        ],
        out_specs=pl.BlockSpec((_BM, D_OUT), lambda i: (i, 0)),
        out_shape=jax.ShapeDtypeStruct((N, D_OUT), jnp.float32),
    )(parts, parts)


@jax.jit
def kernel(x, edge_index, w1, w2):
    ei = edge_index.astype(jnp.int32)
    src = jnp.concatenate([ei[1], jnp.zeros((E_PAD - E,), jnp.int32)])
    dst = jnp.concatenate([ei[0], jnp.full((E_PAD - E,), N, jnp.int32)])
    src = src.reshape(TOT_CHUNKS, CHUNK)
    dst = dst.reshape(TOT_CHUNKS, CHUNK)

    h1 = _mm1(x, w1)
    p1 = _spmm_l1(h1, src, dst)
    h2 = _mid(p1, w2)
    p2 = _spmm_l2(h2, src, dst)
    return _final(p2)


# speedup vs baseline: 1.0379x; 1.0379x over previous
import functools

import jax
import jax.numpy as jnp
from jax import lax
from jax.experimental import pallas as pl
from jax.experimental.pallas import tpu as pltpu
from jax.experimental.pallas import tpu_sc as plsc

N = 10000
E = 320000
D_IN = 128
D_HID = 128
D_OUT = 64

NC = 2
NS = 16
NW = NC * NS

CHUNK = 128
CH = 80
E_PAD = NW * CH * CHUNK
N_ACC = 10240
ROWS_PER_TILE = N_ACC // NS

CB = 40
TOT_CHUNKS = E_PAD // CHUNK
B0 = 3
B1 = 1


def _spmm_body(h_hbm, src_hbm, dst_hbm, out_hbm,
               src_v, dst_v, rows_v, accum, gsem, ssem, zsem, D):
    c = lax.axis_index("c")
    s = lax.axis_index("s")

    nblk = lax.select(c == 0, B0, B1)
    base = lax.select(c == 0, s * (B0 * CB), NS * B0 * CB + s * (B1 * CB))

    def gather_start(q, slot):
        pltpu.async_copy(h_hbm.at[src_v.at[q]], rows_v.at[slot],
                         gsem.at[slot])

    def gather_wait(q, slot):
        pltpu.make_async_copy(h_hbm.at[src_v.at[q]], rows_v.at[slot],
                              gsem.at[slot]).wait()

    def scatter_start(q, slot):
        pltpu.async_copy(rows_v.at[slot], accum.at[dst_v.at[q]],
                         ssem.at[slot], add=True)

    def scatter_wait(q, slot):
        pltpu.make_async_copy(rows_v.at[slot], accum.at[dst_v.at[q]],
                              ssem.at[slot]).wait()

    zvec = jnp.zeros((16,), jnp.float32)
    for i in range(16):
        for jv in range(D // 16):
            rows_v[0, i, pl.ds(jv * 16, 16)] = zvec

    def zero_start(k, carry):
        pltpu.async_copy(rows_v.at[0, pl.ds(0, 16)],
                         accum.at[pl.ds(s * ROWS_PER_TILE + k * 16, 16)],
                         zsem)
        return carry

    def zero_wait(k, carry):
        pltpu.make_async_copy(rows_v.at[0, pl.ds(0, 16)],
                              accum.at[pl.ds(s * ROWS_PER_TILE + k * 16, 16)],
                              zsem).wait()
        return carry

    lax.fori_loop(0, ROWS_PER_TILE // 16, zero_start, 0)
    lax.fori_loop(0, ROWS_PER_TILE // 16, zero_wait, 0)
    plsc.subcore_barrier()

    def blk_body(blk, carry):
        pltpu.sync_copy(src_hbm.at[pl.ds(base + blk * CB, CB)], src_v)
        pltpu.sync_copy(dst_hbm.at[pl.ds(base + blk * CB, CB)], dst_v)

        gather_start(0, 0)
        gather_start(1, 1)
        gather_wait(0, 0)
        scatter_start(0, 0)

        def superstep(s2, carry2):
            for b in range(2):
                q = s2 * 2 + b
                scatter_wait(q - 2, b)
                gather_start(q, b)
                gather_wait(q - 1, 1 - b)
                scatter_start(q - 1, 1 - b)
            return carry2

        lax.fori_loop(1, CB // 2, superstep, 0)

        gather_wait(CB - 1, 1)
        scatter_start(CB - 1, 1)
        scatter_wait(CB - 2, 0)
        scatter_wait(CB - 1, 1)
        return carry

    lax.fori_loop(0, nblk, blk_body, 0)
    plsc.subcore_barrier()

    pltpu.sync_copy(
        accum.at[pl.ds(s * ROWS_PER_TILE, ROWS_PER_TILE)],
        out_hbm.at[c, pl.ds(s * ROWS_PER_TILE, ROWS_PER_TILE)])


def _make_spmm(D):
    mesh = plsc.VectorSubcoreMesh(core_axis_name="c", subcore_axis_name="s")
    body = functools.partial(_spmm_body, D=D)
    return pl.kernel(
        body,
        out_type=jax.ShapeDtypeStruct((NC, N_ACC, D), jnp.float32),
        mesh=mesh,
        scratch_types=[
            pltpu.VMEM((CB, CHUNK), jnp.int32),
            pltpu.VMEM((CB, CHUNK), jnp.int32),
            pltpu.VMEM((2, CHUNK, D), jnp.float32),
            pltpu.VMEM_SHARED((N_ACC, D), jnp.float32),
            pltpu.SemaphoreType.DMA((2,)),
            pltpu.SemaphoreType.DMA((2,)),
            pltpu.SemaphoreType.DMA,
        ],
        compiler_params=pltpu.CompilerParams(use_tc_tiling_on_sc=False),
        name=f"spmm_sc_d{D}",
    )


_spmm_l1 = _make_spmm(D_HID)
_spmm_l2 = _make_spmm(D_OUT)


_BM = 1000


def _mm1_body(x_ref, w_ref, o_ref):
    o_ref[...] = jnp.dot(x_ref[...], w_ref[...],
                         preferred_element_type=jnp.float32)


def _mid_body(p0_ref, p1_ref, w_ref, o_ref):
    h = jnp.maximum(p0_ref[0] + p1_ref[0], 0.0)
    o_ref[...] = jnp.dot(h, w_ref[...], preferred_element_type=jnp.float32)


def _final_body(p0_ref, p1_ref, o_ref):
    z = p0_ref[0] + p1_ref[0]
    m = jnp.max(z, axis=1, keepdims=True)
    lse = jnp.log(jnp.sum(jnp.exp(z - m), axis=1, keepdims=True)) + m
    o_ref[...] = z - lse


def _mm1(x, w1):
    return pl.pallas_call(
        _mm1_body,
        grid=(N // _BM,),
        in_specs=[
            pl.BlockSpec((_BM, D_IN), lambda i: (i, 0)),
            pl.BlockSpec((D_IN, D_HID), lambda i: (0, 0)),
        ],
        out_specs=pl.BlockSpec((_BM, D_HID), lambda i: (i, 0)),
        out_shape=jax.ShapeDtypeStruct((N, D_HID), jnp.float32),
    )(x, w1)


def _mid(parts, w2):
    return pl.pallas_call(
        _mid_body,
        grid=(N // _BM,),
        in_specs=[
            pl.BlockSpec((1, _BM, D_HID), lambda i: (0, i, 0)),
            pl.BlockSpec((1, _BM, D_HID), lambda i: (1, i, 0)),
            pl.BlockSpec((D_HID, D_OUT), lambda i: (0, 0)),
        ],
        out_specs=pl.BlockSpec((_BM, D_OUT), lambda i: (i, 0)),
        out_shape=jax.ShapeDtypeStruct((N, D_OUT), jnp.float32),
    )(parts, parts, w2)


def _final(parts):
    return pl.pallas_call(
        _final_body,
        grid=(N // _BM,),
        in_specs=[
            pl.BlockSpec((1, _BM, D_OUT), lambda i: (0, i, 0)),
            pl.BlockSpec((1, _BM, D_OUT), lambda i: (1, i, 0)),
        ],
        out_specs=pl.BlockSpec((_BM, D_OUT), lambda i: (i, 0)),
        out_shape=jax.ShapeDtypeStruct((N, D_OUT), jnp.float32),
    )(parts, parts)


@jax.jit
def kernel(x, edge_index, w1, w2):
    ei = edge_index.astype(jnp.int32)
    src = jnp.concatenate([ei[1], jnp.zeros((E_PAD - E,), jnp.int32)])
    dst = jnp.concatenate([ei[0], jnp.full((E_PAD - E,), N, jnp.int32)])
    src = src.reshape(TOT_CHUNKS, CHUNK)
    dst = dst.reshape(TOT_CHUNKS, CHUNK)

    h1 = _mm1(x, w1)
    p1 = _spmm_l1(h1, src, dst)
    h2 = _mid(p1, w2)
    p2 = _spmm_l2(h2, src, dst)
    return _final(p2)


# speedup vs baseline: 1.2177x; 1.1732x over previous
import functools

import jax
import jax.numpy as jnp
from jax import lax
from jax.experimental import pallas as pl
from jax.experimental.pallas import tpu as pltpu
from jax.experimental.pallas import tpu_sc as plsc

N = 10000
E = 320000
D_IN = 128
D_HID = 128
D_OUT = 64

NC = 2
NS = 16
NW = NC * NS

CHUNK = 128
CH = 80
E_PAD = NW * CH * CHUNK
N_ACC = 10240
ROWS_PER_TILE = N_ACC // NS

CB = 40
TOT_CHUNKS = E_PAD // CHUNK
B0 = 3
B1 = 1


def _spmm_body(*refs, D, stage_src):
    if stage_src:
        (h_hbm, src_hbm, dst_hbm, out_hbm,
         src_v, dst_v, rows_v, accum, h_sh, gsem, ssem, zsem) = refs
    else:
        (h_hbm, src_hbm, dst_hbm, out_hbm,
         src_v, dst_v, rows_v, accum, gsem, ssem, zsem) = refs
        h_sh = None
    h_src = h_sh if stage_src else h_hbm
    c = lax.axis_index("c")
    s = lax.axis_index("s")

    nblk = lax.select(c == 0, B0, B1)
    base = lax.select(c == 0, s * (B0 * CB), NS * B0 * CB + s * (B1 * CB))

    def gather_start(q, slot):
        pltpu.async_copy(h_src.at[src_v.at[q]], rows_v.at[slot],
                         gsem.at[slot])

    def gather_wait(q, slot):
        pltpu.make_async_copy(h_src.at[src_v.at[q]], rows_v.at[slot],
                              gsem.at[slot]).wait()

    def scatter_start(q, slot):
        pltpu.async_copy(rows_v.at[slot], accum.at[dst_v.at[q]],
                         ssem.at[slot], add=True)

    def scatter_wait(q, slot):
        pltpu.make_async_copy(rows_v.at[slot], accum.at[dst_v.at[q]],
                              ssem.at[slot]).wait()

    zvec = jnp.zeros((16,), jnp.float32)
    for i in range(16):
        for jv in range(D // 16):
            rows_v[0, i, pl.ds(jv * 16, 16)] = zvec

    def zero_start(k, carry):
        pltpu.async_copy(rows_v.at[0, pl.ds(0, 16)],
                         accum.at[pl.ds(s * ROWS_PER_TILE + k * 16, 16)],
                         zsem)
        return carry

    def zero_wait(k, carry):
        pltpu.make_async_copy(rows_v.at[0, pl.ds(0, 16)],
                              accum.at[pl.ds(s * ROWS_PER_TILE + k * 16, 16)],
                              zsem).wait()
        return carry

    lax.fori_loop(0, ROWS_PER_TILE // 16, zero_start, 0)
    if stage_src:
        pltpu.sync_copy(h_hbm.at[pl.ds(s * (N // NS), N // NS)],
                        h_sh.at[pl.ds(s * (N // NS), N // NS)])
    lax.fori_loop(0, ROWS_PER_TILE // 16, zero_wait, 0)
    plsc.subcore_barrier()

    def blk_body(blk, carry):
        pltpu.sync_copy(src_hbm.at[pl.ds(base + blk * CB, CB)], src_v)
        pltpu.sync_copy(dst_hbm.at[pl.ds(base + blk * CB, CB)], dst_v)

        gather_start(0, 0)
        gather_start(1, 1)
        gather_wait(0, 0)
        scatter_start(0, 0)

        def superstep(s2, carry2):
            for b in range(2):
                q = s2 * 2 + b
                scatter_wait(q - 2, b)
                gather_start(q, b)
                gather_wait(q - 1, 1 - b)
                scatter_start(q - 1, 1 - b)
            return carry2

        lax.fori_loop(1, CB // 2, superstep, 0)

        gather_wait(CB - 1, 1)
        scatter_start(CB - 1, 1)
        scatter_wait(CB - 2, 0)
        scatter_wait(CB - 1, 1)
        return carry

    lax.fori_loop(0, nblk, blk_body, 0)
    plsc.subcore_barrier()

    pltpu.sync_copy(
        accum.at[pl.ds(s * ROWS_PER_TILE, ROWS_PER_TILE)],
        out_hbm.at[c, pl.ds(s * ROWS_PER_TILE, ROWS_PER_TILE)])


def _make_spmm(D, stage_src):
    mesh = plsc.VectorSubcoreMesh(core_axis_name="c", subcore_axis_name="s")
    body = functools.partial(_spmm_body, D=D, stage_src=stage_src)
    scratch = [
        pltpu.VMEM((CB, CHUNK), jnp.int32),
        pltpu.VMEM((CB, CHUNK), jnp.int32),
        pltpu.VMEM((2, CHUNK, D), jnp.float32),
        pltpu.VMEM_SHARED((N_ACC, D), jnp.float32),
    ]
    if stage_src:
        scratch.append(pltpu.VMEM_SHARED((N, D), jnp.float32))
    scratch += [
        pltpu.SemaphoreType.DMA((2,)),
        pltpu.SemaphoreType.DMA((2,)),
        pltpu.SemaphoreType.DMA,
    ]
    return pl.kernel(
        body,
        out_type=jax.ShapeDtypeStruct((NC, N_ACC, D), jnp.float32),
        mesh=mesh,
        scratch_types=scratch,
        compiler_params=pltpu.CompilerParams(use_tc_tiling_on_sc=False),
        name=f"spmm_sc_d{D}",
    )


_spmm_l1 = _make_spmm(D_HID, stage_src=False)
_spmm_l2 = _make_spmm(D_OUT, stage_src=True)


_BM = 1000


def _mm1_body(x_ref, w_ref, o_ref):
    o_ref[...] = jnp.dot(x_ref[...], w_ref[...],
                         preferred_element_type=jnp.float32)


def _mid_body(p0_ref, p1_ref, w_ref, o_ref):
    h = jnp.maximum(p0_ref[0] + p1_ref[0], 0.0)
    o_ref[...] = jnp.dot(h, w_ref[...], preferred_element_type=jnp.float32)


def _final_body(p0_ref, p1_ref, o_ref):
    z = p0_ref[0] + p1_ref[0]
    m = jnp.max(z, axis=1, keepdims=True)
    lse = jnp.log(jnp.sum(jnp.exp(z - m), axis=1, keepdims=True)) + m
    o_ref[...] = z - lse


def _mm1(x, w1):
    return pl.pallas_call(
        _mm1_body,
        grid=(N // _BM,),
        in_specs=[
            pl.BlockSpec((_BM, D_IN), lambda i: (i, 0)),
            pl.BlockSpec((D_IN, D_HID), lambda i: (0, 0)),
        ],
        out_specs=pl.BlockSpec((_BM, D_HID), lambda i: (i, 0)),
        out_shape=jax.ShapeDtypeStruct((N, D_HID), jnp.float32),
    )(x, w1)


def _mid(parts, w2):
    return pl.pallas_call(
        _mid_body,
        grid=(N // _BM,),
        in_specs=[
            pl.BlockSpec((1, _BM, D_HID), lambda i: (0, i, 0)),
            pl.BlockSpec((1, _BM, D_HID), lambda i: (1, i, 0)),
            pl.BlockSpec((D_HID, D_OUT), lambda i: (0, 0)),
        ],
        out_specs=pl.BlockSpec((_BM, D_OUT), lambda i: (i, 0)),
        out_shape=jax.ShapeDtypeStruct((N, D_OUT), jnp.float32),
    )(parts, parts, w2)


def _final(parts):
    return pl.pallas_call(
        _final_body,
        grid=(N // _BM,),
        in_specs=[
            pl.BlockSpec((1, _BM, D_OUT), lambda i: (0, i, 0)),
            pl.BlockSpec((1, _BM, D_OUT), lambda i: (1, i, 0)),
        ],
        out_specs=pl.BlockSpec((_BM, D_OUT), lambda i: (i, 0)),
        out_shape=jax.ShapeDtypeStruct((N, D_OUT), jnp.float32),
    )(parts, parts)


@jax.jit
def kernel(x, edge_index, w1, w2):
    ei = edge_index.astype(jnp.int32)
    src = jnp.concatenate([ei[1], jnp.zeros((E_PAD - E,), jnp.int32)])
    dst = jnp.concatenate([ei[0], jnp.full((E_PAD - E,), N, jnp.int32)])
    src = src.reshape(TOT_CHUNKS, CHUNK)
    dst = dst.reshape(TOT_CHUNKS, CHUNK)

    h1 = _mm1(x, w1)
    p1 = _spmm_l1(h1, src, dst)
    h2 = _mid(p1, w2)
    p2 = _spmm_l2(h2, src, dst)
    return _final(p2)
